# baseline (speedup 1.0000x reference)
import functools

import jax
import jax.numpy as jnp
from jax import lax
from jax.experimental import pallas as pl
from jax.experimental.pallas import tpu as pltpu
from jax.experimental.pallas import tpu_sc as plsc

_info = plsc.get_sparse_core_info()
_NC, _NS = _info.num_cores, _info.num_subcores
_NW = _NC * _NS


def _sc_gather_bf16(idx_rs, table, nch, ch, d):
    mesh = plsc.VectorSubcoreMesh(core_axis_name="c", subcore_axis_name="s")
    groups = d // 32

    @functools.partial(
        pl.kernel,
        mesh=mesh,
        out_type=jax.ShapeDtypeStruct((_NW, nch, ch, d // 2), jnp.uint32),
        scratch_types=[
            pltpu.VMEM((nch, ch), jnp.int32),
            pltpu.VMEM((ch, d), jnp.uint32),
            pltpu.VMEM((ch, d), jnp.uint32),
            pltpu.VMEM((ch, d // 2), jnp.uint32),
            pltpu.VMEM((ch, d // 2), jnp.uint32),
            pltpu.SemaphoreType.DMA,
            pltpu.SemaphoreType.DMA,
            pltpu.SemaphoreType.DMA,
            pltpu.SemaphoreType.DMA,
        ],
    )
    def gather_kernel(idx_hbm, table_hbm, out_hbm, idx_v, f0, f1, h0, h1,
                      g0, g1, s0, s1):
        wid = lax.axis_index("s") * _NC + lax.axis_index("c")
        pltpu.sync_copy(idx_hbm.at[wid], idx_v)
        fbufs = (f0, f1)
        hbufs = (h0, h1)
        gsems = (g0, g1)
        ssems = (s0, s1)

        himask = jnp.uint32(0xFFFF0000)

        def start_gather(c, b):
            pltpu.async_copy(table_hbm.at[idx_v.at[c]], fbufs[b], gsems[b])

        def wait_gather(c, b):
            pltpu.make_async_copy(table_hbm.at[idx_v.at[c]], fbufs[b],
                                  gsems[b]).wait()

        def start_store(c, b):
            pltpu.async_copy(hbufs[b], out_hbm.at[wid, c], ssems[b])

        def wait_store(c, b):
            pltpu.make_async_copy(hbufs[b], out_hbm.at[wid, c],
                                  ssems[b]).wait()

        def convert_chunk(b):
            fb, hb = fbufs[b], hbufs[b]

            def row(r, carry):
                for g in range(groups):
                    lo = fb[r, pl.ds(32 * g, 16)]
                    hi = fb[r, pl.ds(32 * g + 16, 16)]
                    hb[r, pl.ds(16 * g, 16)] = (hi & himask) | (lo >> 16)
                return carry

            lax.fori_loop(0, ch, row, 0)

        start_gather(0, 0)

        def pair(i, carry):
            c0 = 2 * i
            c1 = c0 + 1
            wait_gather(c0, 0)
            start_gather(c1, 1)

            @pl.when(i > 0)
            def _():
                wait_store(c0 - 2, 0)

            convert_chunk(0)
            start_store(c0, 0)
            wait_gather(c1, 1)

            @pl.when(i < nch // 2 - 1)
            def _():
                start_gather(c0 + 2, 0)

            @pl.when(i > 0)
            def _():
                wait_store(c1 - 2, 1)

            convert_chunk(1)
            start_store(c1, 1)
            return carry

        lax.fori_loop(0, nch // 2, pair, 0)
        wait_store(nch - 2, 0)
        wait_store(nch - 1, 1)

    return gather_kernel(idx_rs, table)


def _tc_project(x, w, bias2d, n, d, e, bm):
    def mm(x_ref, w_ref, b_ref, o_ref):
        o_ref[...] = lax.dot_general(
            x_ref[...], w_ref[...],
            dimension_numbers=(((1,), (1,)), ((), ())),
            preferred_element_type=jnp.float32,
        ) + b_ref[...]

    return pl.pallas_call(
        mm,
        grid=(n // bm,),
        in_specs=[
            pl.BlockSpec((bm, d), lambda i: (i, 0)),
            pl.BlockSpec((e, d), lambda i: (0, 0)),
            pl.BlockSpec((1, e), lambda i: (0, 0)),
        ],
        out_specs=pl.BlockSpec((bm, e), lambda i: (i, 0)),
        out_shape=jax.ShapeDtypeStruct((n, e), jnp.float32),
    )(x, w, bias2d)


def kernel(idx, tok_emb_table, proj_w, proj_b):
    bsz, t = idx.shape
    v, d = tok_emb_table.shape
    e = proj_w.shape[0]
    n = bsz * t
    n_per_w = n // _NW
    ch = 32
    nch = n_per_w // ch

    idx_rs = idx.reshape(-1).astype(jnp.int32).reshape(_NW, nch, ch)
    table_u32 = lax.bitcast_convert_type(tok_emb_table, jnp.uint32)
    gathered = _sc_gather_bf16(idx_rs, table_u32, nch, ch, d)
    x = lax.bitcast_convert_type(gathered, jnp.bfloat16).reshape(n, d)
    w_perm = proj_w.reshape(e, d // 32, 2, 16).transpose(0, 1, 3, 2)
    w_perm = w_perm.reshape(e, d).astype(jnp.bfloat16)
    y = _tc_project(x, w_perm, proj_b.reshape(1, e), n, d, e, bm=1024)
    return y.reshape(bsz, t, e)


# speedup vs baseline: 8.5043x; 8.5043x over previous
import functools

import jax
import jax.numpy as jnp
from jax import lax
from jax.experimental import pallas as pl
from jax.experimental.pallas import tpu as pltpu
from jax.experimental.pallas import tpu_sc as plsc

_info = plsc.get_sparse_core_info()
_NC, _NS = _info.num_cores, _info.num_subcores
_NW = _NC * _NS
_NBUF = 4


def _sc_gather(idx_rs, table, nch, ch, d):
    mesh = plsc.VectorSubcoreMesh(core_axis_name="c", subcore_axis_name="s")

    @functools.partial(
        pl.kernel,
        mesh=mesh,
        out_type=jax.ShapeDtypeStruct((_NW, nch, ch, d), jnp.float32),
        scratch_types=(
            [pltpu.VMEM((nch, ch), jnp.int32)]
            + [pltpu.VMEM((ch, d), jnp.float32) for _ in range(_NBUF)]
            + [pltpu.SemaphoreType.DMA for _ in range(2 * _NBUF)]
        ),
    )
    def gather_kernel(idx_hbm, table_hbm, out_hbm, idx_v, *bufs_sems):
        bufs = bufs_sems[:_NBUF]
        gsems = bufs_sems[_NBUF:2 * _NBUF]
        ssems = bufs_sems[2 * _NBUF:]
        wid = lax.axis_index("s") * _NC + lax.axis_index("c")
        pltpu.sync_copy(idx_hbm.at[wid], idx_v)

        def start_gather(c):
            b = c % _NBUF
            return pltpu.async_copy(table_hbm.at[idx_v.at[c]], bufs[b],
                                    gsems[b])

        def start_store(c):
            b = c % _NBUF
            return pltpu.async_copy(bufs[b], out_hbm.at[wid, c], ssems[b])

        gd = [None] * nch
        sd = [None] * nch
        gd[0] = start_gather(0)
        gd[1] = start_gather(1)
        for c in range(nch):
            gd[c].wait()
            sd[c] = start_store(c)
            nc = c + 2
            if nc < nch:
                if nc - _NBUF >= 0:
                    sd[nc - _NBUF].wait()
                gd[nc] = start_gather(nc)
        for c in range(max(0, nch - _NBUF), nch):
            sd[c].wait()

    return gather_kernel(idx_rs, table)


def _tc_project(x, w, bias2d, n, d, e, bm):
    def mm(x_ref, w_ref, b_ref, o_ref):
        o_ref[...] = lax.dot_general(
            x_ref[...], w_ref[...],
            dimension_numbers=(((1,), (1,)), ((), ())),
            preferred_element_type=jnp.float32,
        ) + b_ref[...]

    return pl.pallas_call(
        mm,
        grid=(n // bm,),
        in_specs=[
            pl.BlockSpec((bm, d), lambda i: (i, 0)),
            pl.BlockSpec((e, d), lambda i: (0, 0)),
            pl.BlockSpec((1, e), lambda i: (0, 0)),
        ],
        out_specs=pl.BlockSpec((bm, e), lambda i: (i, 0)),
        out_shape=jax.ShapeDtypeStruct((n, e), jnp.float32),
    )(x, w, bias2d)


def kernel(idx, tok_emb_table, proj_w, proj_b):
    bsz, t = idx.shape
    v, d = tok_emb_table.shape
    e = proj_w.shape[0]
    n = bsz * t
    n_per_w = n // _NW
    ch = 32
    nch = n_per_w // ch

    idx_rs = idx.reshape(-1).astype(jnp.int32).reshape(_NW, nch, ch)
    gathered = _sc_gather(idx_rs, tok_emb_table, nch, ch, d)
    x = gathered.reshape(n, d)
    y = _tc_project(x, proj_w, proj_b.reshape(1, e), n, d, e, bm=2048)
    return y.reshape(bsz, t, e)


# speedup vs baseline: 8.6207x; 1.0137x over previous
import functools

import jax
import jax.numpy as jnp
from jax import lax
from jax.experimental import pallas as pl
from jax.experimental.pallas import tpu as pltpu
from jax.experimental.pallas import tpu_sc as plsc

_info = plsc.get_sparse_core_info()
_NC, _NS = _info.num_cores, _info.num_subcores
_NW = _NC * _NS
_NBUF = 4


def _sc_gather(idx_rs, table, nch, ch, d):
    mesh = plsc.VectorSubcoreMesh(core_axis_name="c", subcore_axis_name="s")

    @functools.partial(
        pl.kernel,
        mesh=mesh,
        out_type=jax.ShapeDtypeStruct((_NW, nch, ch, d), jnp.float32),
        scratch_types=(
            [pltpu.VMEM((nch, ch), jnp.int32)]
            + [pltpu.VMEM((ch, d), jnp.float32) for _ in range(_NBUF)]
            + [pltpu.SemaphoreType.DMA for _ in range(2 * _NBUF)]
        ),
    )
    def gather_kernel(idx_hbm, table_hbm, out_hbm, idx_v, *bufs_sems):
        bufs = bufs_sems[:_NBUF]
        gsems = bufs_sems[_NBUF:2 * _NBUF]
        ssems = bufs_sems[2 * _NBUF:]
        wid = lax.axis_index("s") * _NC + lax.axis_index("c")
        pltpu.sync_copy(idx_hbm.at[wid], idx_v)

        def start_gather(c):
            b = c % _NBUF
            return pltpu.async_copy(table_hbm.at[idx_v.at[c]], bufs[b],
                                    gsems[b])

        def start_store(c):
            b = c % _NBUF
            return pltpu.async_copy(bufs[b], out_hbm.at[wid, c], ssems[b])

        gd = [None] * nch
        sd = [None] * nch
        gd[0] = start_gather(0)
        gd[1] = start_gather(1)
        for c in range(nch):
            gd[c].wait()
            sd[c] = start_store(c)
            nc = c + 2
            if nc < nch:
                if nc - _NBUF >= 0:
                    sd[nc - _NBUF].wait()
                gd[nc] = start_gather(nc)
        for c in range(max(0, nch - _NBUF), nch):
            sd[c].wait()

    return gather_kernel(idx_rs, table)


def _tc_project(x, w, bias2d, n, d, e, bm):
    def mm(x_ref, w_ref, b_ref, o_ref):
        o_ref[...] = lax.dot_general(
            x_ref[...], w_ref[...],
            dimension_numbers=(((1,), (1,)), ((), ())),
            preferred_element_type=jnp.float32,
        ) + b_ref[...]

    return pl.pallas_call(
        mm,
        grid=(n // bm,),
        in_specs=[
            pl.BlockSpec((bm, d), lambda i: (i, 0)),
            pl.BlockSpec((e, d), lambda i: (0, 0)),
            pl.BlockSpec((1, e), lambda i: (0, 0)),
        ],
        out_specs=pl.BlockSpec((bm, e), lambda i: (i, 0)),
        out_shape=jax.ShapeDtypeStruct((n, e), jnp.float32),
    )(x, w, bias2d)


def kernel(idx, tok_emb_table, proj_w, proj_b):
    bsz, t = idx.shape
    v, d = tok_emb_table.shape
    e = proj_w.shape[0]
    n = bsz * t
    n_per_w = n // _NW
    ch = 32
    nch = n_per_w // ch

    idx_rs = idx.reshape(-1).astype(jnp.int32).reshape(_NW, nch, ch)
    gathered = _sc_gather(idx_rs, tok_emb_table, nch, ch, d)
    x = gathered.reshape(n, d)
    y = _tc_project(x, proj_w, proj_b.reshape(1, e), n, d, e, bm=4096)
    return y.reshape(bsz, t, e)
